# Initial kernel scaffold; baseline (speedup 1.0000x reference)
import jax
import jax.numpy as jnp
from jax.experimental import pallas as pl


def kernel(x, edge_index, batch, W1, b1, W2, b2, Wl, bl):
    raise NotImplementedError("write your pallas kernel here")


# speedup vs baseline: 16.2401x; 16.2401x over previous
import functools

import jax
import jax.numpy as jnp
from jax import lax
from jax.experimental import pallas as pl
from jax.experimental.pallas import tpu as pltpu
from jax.experimental.pallas import tpu_sc as plsc

N = 10000
E = 320000
D = 128
G = 128

NC = 2
NS = 16
NW = NC * NS
EPW = E // NW
CH = 128
NFULL = EPW // CH
TAIL = EPW - NFULL * CH
NP = 10240
RPW = NP // NS
RB = 1000
NB = N // RB

_mesh = plsc.VectorSubcoreMesh(core_axis_name="c", subcore_axis_name="s")


NR = NP // 128

@functools.partial(
    pl.kernel,
    out_type=jax.ShapeDtypeStruct((NC, NP, 16), jnp.float32),
    mesh=_mesh,
    scratch_types=[
        pltpu.VMEM_SHARED((NR, 128), jnp.float32),
        pltpu.VMEM((CH,), jnp.int32),
        pltpu.VMEM((NR, 128), jnp.float32),
        pltpu.VMEM((TAIL,), jnp.int32),
        pltpu.VMEM((NR,), jnp.int32),
        pltpu.VMEM((RPW, 16), jnp.float32),
    ],
    compiler_params=pltpu.CompilerParams(needs_layout_passes=False),
)
def _deg_kernel(dst_hbm, z80_hbm, i80_hbm, out_hbm,
                accs, idxd, loc, idxt, rowid, outcol):
    c = lax.axis_index("c")
    s = lax.axis_index("s")
    wid = c * NS + s

    @pl.when(s == 0)
    def _zero_shared():
        pltpu.sync_copy(z80_hbm, accs)

    pltpu.sync_copy(z80_hbm, loc)
    pltpu.sync_copy(i80_hbm, rowid)
    plsc.subcore_barrier()
    base = wid * EPW
    lane = lax.broadcasted_iota(jnp.int32, (16,), 0)

    def group(d16):
        cnt, last = plsc.scan_count(d16)
        plsc.addupdate_scatter(loc, [d16 >> 7, d16 & 127],
                               cnt.astype(jnp.float32), mask=last)

    def step(k, carry):
        pltpu.sync_copy(dst_hbm.at[pl.ds(base + k * CH, CH)], idxd)
        for t in range(CH // 16):
            group(idxd[pl.ds(t * 16, 16)])
        return carry

    lax.fori_loop(0, NFULL, step, 0)
    pltpu.sync_copy(dst_hbm.at[pl.ds(base + NFULL * CH, TAIL)], idxt)
    for t in range(TAIL // 16):
        group(idxt[pl.ds(t * 16, 16)])
    pltpu.sync_copy(loc, accs.at[rowid], add=True)
    plsc.subcore_barrier()
    pltpu.sync_copy(accs, loc)
    vbase = s * RPW
    for j in range(RPW // 16):
        v16 = vbase + j * 16 + lane
        deg16 = plsc.load_gather(loc, [v16 >> 7, v16 & 127])
        plsc.store_scatter(outcol, [j * 16 + lane, lane * 0], deg16)
    pltpu.sync_copy(outcol, out_hbm.at[c, pl.ds(vbase, RPW)])


@functools.partial(
    pl.kernel,
    out_type=jax.ShapeDtypeStruct((NC, NP, D), jnp.float32),
    mesh=_mesh,
    scratch_types=[
        pltpu.VMEM_SHARED((NP, D), jnp.float32),
        pltpu.VMEM((CH,), jnp.int32),
        pltpu.VMEM((CH,), jnp.int32),
        pltpu.VMEM((CH, D), jnp.float32),
        pltpu.VMEM((TAIL,), jnp.int32),
        pltpu.VMEM((TAIL,), jnp.int32),
        pltpu.VMEM((TAIL, D), jnp.float32),
        pltpu.SemaphoreType.DMA,
    ],
)
def _agg_kernel(g_hbm, src_hbm, dst_hbm, zrow_hbm, out_hbm,
                acc, idxs, idxd, rows, idxs_t, idxd_t, rows_t, sem):
    c = lax.axis_index("c")
    s = lax.axis_index("s")
    wid = c * NS + s
    pltpu.sync_copy(zrow_hbm, acc.at[pl.ds(s * RPW, RPW)])
    plsc.subcore_barrier()
    base = wid * EPW

    def step(k, carry):
        off = base + k * CH
        pltpu.sync_copy(src_hbm.at[pl.ds(off, CH)], idxs)
        pltpu.sync_copy(dst_hbm.at[pl.ds(off, CH)], idxd)
        pltpu.async_copy(g_hbm.at[idxs], rows, sem).wait()
        pltpu.sync_copy(rows, acc.at[idxd], add=True)
        return carry

    lax.fori_loop(0, NFULL, step, 0)
    off = base + NFULL * CH
    pltpu.sync_copy(src_hbm.at[pl.ds(off, TAIL)], idxs_t)
    pltpu.sync_copy(dst_hbm.at[pl.ds(off, TAIL)], idxd_t)
    pltpu.async_copy(g_hbm.at[idxs_t], rows_t, sem).wait()
    pltpu.sync_copy(rows_t, acc.at[idxd_t], add=True)
    plsc.subcore_barrier()
    pltpu.sync_copy(acc.at[pl.ds(s * RPW, RPW)],
                    out_hbm.at[c, pl.ds(s * RPW, RPW)])


def _dinv_block(deg_block):
    deg = deg_block[0, :, 0:1] + deg_block[1, :, 0:1] + 1.0
    return lax.rsqrt(deg)


def _tc1_body(deg_ref, x_ref, w1_ref, g1_ref):
    dinv = _dinv_block(deg_ref[...])
    h = jnp.dot(x_ref[...], w1_ref[...], preferred_element_type=jnp.float32)
    g1_ref[...] = h * dinv


def _tc2_body(deg_ref, p_ref, g1_ref, b1_ref, w2_ref, g2_ref):
    dinv = _dinv_block(deg_ref[...])
    p = p_ref[...]
    z = jnp.maximum((p[0] + p[1] + g1_ref[...]) * dinv + b1_ref[...], 0.0)
    h2 = jnp.dot(z, w2_ref[...], preferred_element_type=jnp.float32)
    g2_ref[...] = h2 * dinv


def _tc3_body(deg_ref, p_ref, g2_ref, b2_ref, bt_ref, wl_ref, bl_ref,
              out_ref, pool_scr, cnt_scr):
    i = pl.program_id(0)

    @pl.when(i == 0)
    def _init():
        pool_scr[...] = jnp.zeros_like(pool_scr)
        cnt_scr[...] = jnp.zeros_like(cnt_scr)

    dinv = _dinv_block(deg_ref[...])
    p = p_ref[...]
    z = (p[0] + p[1] + g2_ref[...]) * dinv + b2_ref[...]
    seg = bt_ref[...].reshape(RB, 1)
    gid = lax.broadcasted_iota(jnp.int32, (RB, G), 1).astype(jnp.float32)
    m = (seg == gid).astype(jnp.float32)
    dn = (((0,), (0,)), ((), ()))
    pool_scr[...] += lax.dot_general(m, z, dn,
                                     preferred_element_type=jnp.float32)
    cnt_scr[...] += lax.dot_general(m, jnp.ones((RB, G), jnp.float32), dn,
                                    preferred_element_type=jnp.float32)

    @pl.when(i == pl.num_programs(0) - 1)
    def _fin():
        mean = pool_scr[...] / jnp.maximum(cnt_scr[...], 1.0)
        out_ref[...] = (jnp.dot(mean, wl_ref[...],
                                preferred_element_type=jnp.float32)
                        + bl_ref[...])


_deg_spec = pl.BlockSpec((NC, RB, 16), lambda i: (0, i, 0))
_row_spec = pl.BlockSpec((RB, D), lambda i: (i, 0))
_part_spec = pl.BlockSpec((NC, RB, D), lambda i: (0, i, 0))
_mat_spec = pl.BlockSpec((D, D), lambda i: (0, 0))
_vec_spec = pl.BlockSpec((1, D), lambda i: (0, 0))

_tc1 = pl.pallas_call(
    _tc1_body,
    grid=(NB,),
    in_specs=[_deg_spec, _row_spec, _mat_spec],
    out_specs=_row_spec,
    out_shape=jax.ShapeDtypeStruct((N, D), jnp.float32),
)

_tc2 = pl.pallas_call(
    _tc2_body,
    grid=(NB,),
    in_specs=[_deg_spec, _part_spec, _row_spec, _vec_spec, _mat_spec],
    out_specs=_row_spec,
    out_shape=jax.ShapeDtypeStruct((N, D), jnp.float32),
)

_tc3 = pl.pallas_call(
    _tc3_body,
    grid=(NB,),
    in_specs=[
        _deg_spec, _part_spec, _row_spec, _vec_spec,
        pl.BlockSpec((1, 1, RB), lambda i: (i, 0, 0)),
        _mat_spec, _vec_spec,
    ],
    out_specs=pl.BlockSpec((G, D), lambda i: (0, 0)),
    out_shape=jax.ShapeDtypeStruct((G, D), jnp.float32),
    scratch_shapes=[
        pltpu.VMEM((G, D), jnp.float32),
        pltpu.VMEM((G, G), jnp.float32),
    ],
)


def kernel(x, edge_index, batch, W1, b1, W2, b2, Wl, bl):
    src = edge_index[0]
    dst = edge_index[1]

    z80 = jnp.zeros((NR, 128), jnp.float32)
    i80 = jnp.arange(NR, dtype=jnp.int32)
    zrow = jnp.zeros((RPW, D), jnp.float32)
    b1r = b1.reshape(1, D)
    b2r = b2.reshape(1, D)
    wlp = jnp.zeros((D, D), jnp.float32).at[:, :Wl.shape[1]].set(Wl)
    blp = jnp.zeros((1, D), jnp.float32).at[0, :bl.shape[0]].set(bl)
    batf = batch.astype(jnp.float32).reshape(NB, 1, RB)

    degp = _deg_kernel(dst, z80, i80)
    g1 = _tc1(degp, x, W1)
    p1 = _agg_kernel(g1, src, dst, zrow)
    g2 = _tc2(degp, p1, g1, b1r, W2)
    p2 = _agg_kernel(g2, src, dst, zrow)
    outp = _tc3(degp, p2, g2, b2r, batf, wlp, blp)
    return outp[:, :Wl.shape[1]]
